# baseline (speedup 1.0000x reference)
import functools
import math

import jax
import jax.numpy as jnp
from jax import lax
from jax.experimental import pallas as pl
from jax.experimental.pallas import tpu as pltpu
from jax.experimental.pallas import tpu_sc as plsc

N_NODES = 100000
N_EDGES = 3200000

NC = 2
NS = 16
NW = NC * NS
EPW = N_EDGES // NW
CH = 2000
NCHUNK = EPW // CH
CHB = 4000
NCHUNKB = EPW // CHB
N_PAD = 100096
NPT = N_PAD // NS


def _pass_a(*refs):
    (node_t_hbm, src_hbm, h_hbm, r_hbm, w_hbm, y_hbm) = refs[:6]
    src_v, h_v, rbuf_v = refs[6:9]
    ridx = refs[9:12]
    r_v = refs[12:15]
    x_v = refs[15:23]
    y_v = refs[23:31]
    w_v, stage_v = refs[31:33]
    nt = refs[33:41]
    rb_sh = refs[41]
    gsem = refs[42]

    cid = lax.axis_index("c")
    sid = lax.axis_index("s")
    wid = cid * NS + sid

    rowbase = sid * NPT
    for c in range(8):
        pltpu.sync_copy(node_t_hbm.at[pl.ds(c * N_PAD + rowbase, NPT)],
                        stage_v)
        pltpu.sync_copy(stage_v, nt[c].at[pl.ds(rowbase, NPT)])

    pltpu.sync_copy(w_hbm, w_v)

    lanes16 = lax.iota(jnp.int32, 16)
    shbase = sid * (3 * CH)

    def idxfill_body(i, _):
        j16 = i * 16 + lanes16
        for c in range(3):
            ridx[c][pl.ds(i * 16, 16)] = shbase + 3 * j16 + c
        return 0

    lax.fori_loop(0, CH // 16, idxfill_body, 0)
    plsc.subcore_barrier()

    w_lo = w_v[pl.ds(0, 16)]
    w_hi = w_v[pl.ds(16, 16)]
    (a00, a10, d00, d10, a01, a11, d01, d11,
     b00, b10, f00, f10, b01, b11, f01, f11) = [w_lo[k] for k in range(16)]
    c00, c10, c01, c11 = [w_hi[k] for k in range(4)]

    def chunk_body(cc, _):
        base = wid * EPW + cc * CH
        pltpu.sync_copy(src_hbm.at[pl.ds(base, CH)], src_v)
        pltpu.sync_copy(h_hbm.at[pl.ds(base, CH)], h_v)
        pltpu.sync_copy(r_hbm.at[pl.ds(3 * base, 3 * CH)], rbuf_v)
        pltpu.sync_copy(rbuf_v, rb_sh.at[pl.ds(shbase, 3 * CH)])
        descs = [pltpu.async_copy(nt[c].at[src_v], x_v[c], gsem)
                 for c in range(8)]
        descs += [pltpu.async_copy(rb_sh.at[ridx[c]], r_v[c], gsem)
                  for c in range(3)]
        for d in descs:
            d.wait()

        def edge_body(i, _):
            sl = pl.ds(i * 16, 16)
            h = h_v[sl]
            r0, r1, r2 = r_v[0][sl], r_v[1][sl], r_v[2][sl]
            a0, a1 = x_v[0][sl], x_v[1][sl]
            u0, u1, u2 = x_v[2][sl], x_v[3][sl], x_v[4][sl]
            v0, v1, v2 = x_v[5][sl], x_v[6][sl], x_v[7][sl]

            t0 = h * a0
            t1 = h * a1
            dot_u = r0 * u0 + r1 * u1 + r2 * u2
            dot_v = r0 * v0 + r1 * v1 + r2 * v2
            y_v[0][sl] = a00 * t0 + a10 * t1 + d00 * dot_u + d10 * dot_v
            y_v[1][sl] = a01 * t0 + a11 * t1 + d01 * dot_u + d11 * dot_v

            hu0, hu1, hu2 = h * u0, h * u1, h * u2
            hv0, hv1, hv2 = h * v0, h * v1, h * v2
            ca = c00 * a0 + c10 * a1
            cb = c01 * a0 + c11 * a1
            cu0 = r1 * u2 - r2 * u1
            cu1 = r2 * u0 - r0 * u2
            cu2 = r0 * u1 - r1 * u0
            cv0 = r1 * v2 - r2 * v1
            cv1 = r2 * v0 - r0 * v2
            cv2 = r0 * v1 - r1 * v0

            y_v[2][sl] = b00 * hu0 + b10 * hv0 + ca * r0 + f00 * cu0 + f10 * cv0
            y_v[3][sl] = b00 * hu1 + b10 * hv1 + ca * r1 + f00 * cu1 + f10 * cv1
            y_v[4][sl] = b00 * hu2 + b10 * hv2 + ca * r2 + f00 * cu2 + f10 * cv2
            y_v[5][sl] = b01 * hu0 + b11 * hv0 + cb * r0 + f01 * cu0 + f11 * cv0
            y_v[6][sl] = b01 * hu1 + b11 * hv1 + cb * r1 + f01 * cu1 + f11 * cv1
            y_v[7][sl] = b01 * hu2 + b11 * hv2 + cb * r2 + f01 * cu2 + f11 * cv2
            return 0

        lax.fori_loop(0, CH // 16, edge_body, 0)
        for c in range(8):
            pltpu.sync_copy(y_v[c], y_hbm.at[pl.ds(c * N_EDGES + base, CH)])
        return 0

    lax.fori_loop(0, NCHUNK, chunk_body, 0)


_pass_a_call = functools.partial(
    pl.kernel,
    out_type=jax.ShapeDtypeStruct((8 * N_EDGES,), jnp.float32),
    mesh=plsc.VectorSubcoreMesh(core_axis_name="c", subcore_axis_name="s"),
    scratch_types=(
        [pltpu.VMEM((CH,), jnp.int32)]
        + [pltpu.VMEM((CH,), jnp.float32)]
        + [pltpu.VMEM((3 * CH,), jnp.float32)]
        + [pltpu.VMEM((CH,), jnp.int32)] * 3
        + [pltpu.VMEM((CH,), jnp.float32)] * 3
        + [pltpu.VMEM((CH,), jnp.float32)] * 8
        + [pltpu.VMEM((CH,), jnp.float32)] * 8
        + [pltpu.VMEM((32,), jnp.float32)]
        + [pltpu.VMEM((NPT,), jnp.float32)]
        + [pltpu.VMEM_SHARED((N_PAD,), jnp.float32)] * 8
        + [pltpu.VMEM_SHARED((NS * 3 * CH,), jnp.float32)]
        + [pltpu.SemaphoreType.DMA]
    ),
)(_pass_a)


def _pass_b(*refs):
    (tgt_hbm, y_hbm, out0_hbm, out1_hbm) = refs[:4]
    tgt_v = refs[4]
    y_v = refs[5:13]
    stage_v = refs[13]
    acc = refs[14:22]

    cid = lax.axis_index("c")
    sid = lax.axis_index("s")
    wid = cid * NS + sid
    rowbase = sid * NPT

    def zfill_body(i, _):
        stage_v[pl.ds(i * 16, 16)] = jnp.zeros((16,), jnp.float32)
        return 0

    lax.fori_loop(0, NPT // 16, zfill_body, 0)
    for c in range(8):
        pltpu.sync_copy(stage_v, acc[c].at[pl.ds(rowbase, NPT)])
    plsc.subcore_barrier()

    def chunk_body(cc, _):
        base = wid * EPW + cc * CHB
        pltpu.sync_copy(tgt_hbm.at[pl.ds(N_EDGES + base, CHB)], tgt_v)
        for c in range(8):
            pltpu.sync_copy(y_hbm.at[pl.ds(c * N_EDGES + base, CHB)], y_v[c])
        for c in range(8):
            pltpu.sync_copy(y_v[c], acc[c].at[tgt_v], add=True)
        return 0

    lax.fori_loop(0, NCHUNKB, chunk_body, 0)
    plsc.subcore_barrier()

    for c in range(8):
        pltpu.sync_copy(acc[c].at[pl.ds(rowbase, NPT)], stage_v)

        @pl.when(cid == 0)
        def _(c=c):
            pltpu.sync_copy(stage_v,
                            out0_hbm.at[pl.ds(c * N_PAD + rowbase, NPT)])

        @pl.when(cid == 1)
        def _(c=c):
            pltpu.sync_copy(stage_v,
                            out1_hbm.at[pl.ds(c * N_PAD + rowbase, NPT)])


_pass_b_call = functools.partial(
    pl.kernel,
    out_type=(jax.ShapeDtypeStruct((8 * N_PAD,), jnp.float32),
              jax.ShapeDtypeStruct((8 * N_PAD,), jnp.float32)),
    mesh=plsc.VectorSubcoreMesh(core_axis_name="c", subcore_axis_name="s"),
    scratch_types=(
        [pltpu.VMEM((CHB,), jnp.int32)]
        + [pltpu.VMEM((CHB,), jnp.float32)] * 8
        + [pltpu.VMEM((NPT,), jnp.float32)]
        + [pltpu.VMEM_SHARED((N_PAD,), jnp.float32)] * 8
    ),
)(_pass_b)


def _sum_t_body(a_ref, b_ref, o_ref):
    o_ref[...] = jnp.transpose(a_ref[...] + b_ref[...])


def _tc_sum_t(p0, p1):
    tcw = 5888
    return pl.pallas_call(
        _sum_t_body,
        grid=(N_PAD // tcw,),
        in_specs=[pl.BlockSpec((8, tcw), lambda i: (0, i)),
                  pl.BlockSpec((8, tcw), lambda i: (0, i))],
        out_specs=pl.BlockSpec((tcw, 8), lambda i: (i, 0)),
        out_shape=jax.ShapeDtypeStruct((N_PAD, 8), jnp.float32),
    )(p0, p1)


@jax.jit
def kernel(node_irreps, edge_index, sh_edge_features_0, sh_edge_features_1, W):
    edge_flat = edge_index.reshape(2 * N_EDGES)
    h = sh_edge_features_0.reshape(N_EDGES)
    r_flat = sh_edge_features_1.reshape(3 * N_EDGES)
    node_t = jnp.pad(node_irreps, ((0, N_PAD - N_NODES), (0, 0))).T

    s3 = 1.0 / math.sqrt(3.0)
    s6 = 1.0 / math.sqrt(6.0)
    A, B, C, D, F = W[0], W[1] * s3, W[2] * s3, W[3] * s3, W[4] * s6
    wflat = jnp.concatenate([
        jnp.stack([A[0, 0], A[1, 0], D[0, 0], D[1, 0],
                   A[0, 1], A[1, 1], D[0, 1], D[1, 1],
                   B[0, 0], B[1, 0], F[0, 0], F[1, 0],
                   B[0, 1], B[1, 1], F[0, 1], F[1, 1],
                   C[0, 0], C[1, 0], C[0, 1], C[1, 1]]),
        jnp.zeros((12,), jnp.float32),
    ])

    y = _pass_a_call(node_t.reshape(8 * N_PAD), edge_flat, h, r_flat, wflat)
    p0, p1 = _pass_b_call(edge_flat, y)
    return _tc_sum_t(p0.reshape(8, N_PAD), p1.reshape(8, N_PAD))[:N_NODES]


# speedup vs baseline: 9.8991x; 9.8991x over previous
import functools
import math

import jax
import jax.numpy as jnp
from jax import lax
from jax.experimental import pallas as pl
from jax.experimental.pallas import tpu as pltpu
from jax.experimental.pallas import tpu_sc as plsc

N_NODES = 100000
N_EDGES = 3200000

NC = 2
NS = 16
NW = NC * NS
EPW = N_EDGES // NW
CH = 2000
NCHUNK = EPW // CH
CHB = 4000
NCHUNKB = EPW // CHB
N_PAD = 100096
NPT = N_PAD // NS


def _pass_a(*refs):
    (node_t_hbm, src_hbm, h_hbm, r0_hbm, r1_hbm, r2_hbm, w_hbm,
     y_hbm) = refs[:8]
    src_v, h_v = refs[8:10]
    r_v = refs[10:13]
    x_v = refs[13:21]
    y_v = refs[21:29]
    w_v, stage_v = refs[29:31]
    nt = refs[31:39]
    gsem = refs[39]

    cid = lax.axis_index("c")
    sid = lax.axis_index("s")
    wid = cid * NS + sid

    rowbase = sid * NPT
    for c in range(8):
        pltpu.sync_copy(node_t_hbm.at[pl.ds(c * N_PAD + rowbase, NPT)],
                        stage_v)
        pltpu.sync_copy(stage_v, nt[c].at[pl.ds(rowbase, NPT)])

    pltpu.sync_copy(w_hbm, w_v)
    plsc.subcore_barrier()

    w_lo = w_v[pl.ds(0, 16)]
    w_hi = w_v[pl.ds(16, 16)]
    (a00, a10, d00, d10, a01, a11, d01, d11,
     b00, b10, f00, f10, b01, b11, f01, f11) = [w_lo[k] for k in range(16)]
    c00, c10, c01, c11 = [w_hi[k] for k in range(4)]

    def chunk_body(cc, _):
        base = wid * EPW + cc * CH
        pltpu.sync_copy(src_hbm.at[pl.ds(base, CH)], src_v)
        pltpu.sync_copy(h_hbm.at[pl.ds(base, CH)], h_v)
        pltpu.sync_copy(r0_hbm.at[pl.ds(base, CH)], r_v[0])
        pltpu.sync_copy(r1_hbm.at[pl.ds(base, CH)], r_v[1])
        pltpu.sync_copy(r2_hbm.at[pl.ds(base, CH)], r_v[2])
        descs = [pltpu.async_copy(nt[c].at[src_v], x_v[c], gsem)
                 for c in range(8)]
        for d in descs:
            d.wait()

        def edge_body(i, _):
            sl = pl.ds(i * 16, 16)
            h = h_v[sl]
            r0, r1, r2 = r_v[0][sl], r_v[1][sl], r_v[2][sl]
            a0, a1 = x_v[0][sl], x_v[1][sl]
            u0, u1, u2 = x_v[2][sl], x_v[3][sl], x_v[4][sl]
            v0, v1, v2 = x_v[5][sl], x_v[6][sl], x_v[7][sl]

            t0 = h * a0
            t1 = h * a1
            dot_u = r0 * u0 + r1 * u1 + r2 * u2
            dot_v = r0 * v0 + r1 * v1 + r2 * v2
            y_v[0][sl] = a00 * t0 + a10 * t1 + d00 * dot_u + d10 * dot_v
            y_v[1][sl] = a01 * t0 + a11 * t1 + d01 * dot_u + d11 * dot_v

            hu0, hu1, hu2 = h * u0, h * u1, h * u2
            hv0, hv1, hv2 = h * v0, h * v1, h * v2
            ca = c00 * a0 + c10 * a1
            cb = c01 * a0 + c11 * a1
            cu0 = r1 * u2 - r2 * u1
            cu1 = r2 * u0 - r0 * u2
            cu2 = r0 * u1 - r1 * u0
            cv0 = r1 * v2 - r2 * v1
            cv1 = r2 * v0 - r0 * v2
            cv2 = r0 * v1 - r1 * v0

            y_v[2][sl] = b00 * hu0 + b10 * hv0 + ca * r0 + f00 * cu0 + f10 * cv0
            y_v[3][sl] = b00 * hu1 + b10 * hv1 + ca * r1 + f00 * cu1 + f10 * cv1
            y_v[4][sl] = b00 * hu2 + b10 * hv2 + ca * r2 + f00 * cu2 + f10 * cv2
            y_v[5][sl] = b01 * hu0 + b11 * hv0 + cb * r0 + f01 * cu0 + f11 * cv0
            y_v[6][sl] = b01 * hu1 + b11 * hv1 + cb * r1 + f01 * cu1 + f11 * cv1
            y_v[7][sl] = b01 * hu2 + b11 * hv2 + cb * r2 + f01 * cu2 + f11 * cv2
            return 0

        lax.fori_loop(0, CH // 16, edge_body, 0)
        for c in range(8):
            pltpu.sync_copy(y_v[c], y_hbm.at[pl.ds(c * N_EDGES + base, CH)])
        return 0

    lax.fori_loop(0, NCHUNK, chunk_body, 0)


_pass_a_call = functools.partial(
    pl.kernel,
    out_type=jax.ShapeDtypeStruct((8 * N_EDGES,), jnp.float32),
    mesh=plsc.VectorSubcoreMesh(core_axis_name="c", subcore_axis_name="s"),
    scratch_types=(
        [pltpu.VMEM((CH,), jnp.int32)]
        + [pltpu.VMEM((CH,), jnp.float32)]
        + [pltpu.VMEM((CH,), jnp.float32)] * 3
        + [pltpu.VMEM((CH,), jnp.float32)] * 8
        + [pltpu.VMEM((CH,), jnp.float32)] * 8
        + [pltpu.VMEM((32,), jnp.float32)]
        + [pltpu.VMEM((NPT,), jnp.float32)]
        + [pltpu.VMEM_SHARED((N_PAD,), jnp.float32)] * 8
        + [pltpu.SemaphoreType.DMA]
    ),
)(_pass_a)


def _pass_b(*refs):
    (tgt_hbm, y_hbm, out0_hbm, out1_hbm) = refs[:4]
    tgt_v = refs[4]
    y_v = refs[5:13]
    stage_v = refs[13]
    acc = refs[14:22]

    cid = lax.axis_index("c")
    sid = lax.axis_index("s")
    wid = cid * NS + sid
    rowbase = sid * NPT

    def zfill_body(i, _):
        stage_v[pl.ds(i * 16, 16)] = jnp.zeros((16,), jnp.float32)
        return 0

    lax.fori_loop(0, NPT // 16, zfill_body, 0)
    for c in range(8):
        pltpu.sync_copy(stage_v, acc[c].at[pl.ds(rowbase, NPT)])
    plsc.subcore_barrier()

    def chunk_body(cc, _):
        base = wid * EPW + cc * CHB
        pltpu.sync_copy(tgt_hbm.at[pl.ds(N_EDGES + base, CHB)], tgt_v)
        for c in range(8):
            pltpu.sync_copy(y_hbm.at[pl.ds(c * N_EDGES + base, CHB)], y_v[c])
        for c in range(8):
            pltpu.sync_copy(y_v[c], acc[c].at[tgt_v], add=True)
        return 0

    lax.fori_loop(0, NCHUNKB, chunk_body, 0)
    plsc.subcore_barrier()

    for c in range(8):
        pltpu.sync_copy(acc[c].at[pl.ds(rowbase, NPT)], stage_v)

        @pl.when(cid == 0)
        def _(c=c):
            pltpu.sync_copy(stage_v,
                            out0_hbm.at[pl.ds(c * N_PAD + rowbase, NPT)])

        @pl.when(cid == 1)
        def _(c=c):
            pltpu.sync_copy(stage_v,
                            out1_hbm.at[pl.ds(c * N_PAD + rowbase, NPT)])


_pass_b_call = functools.partial(
    pl.kernel,
    out_type=(jax.ShapeDtypeStruct((8 * N_PAD,), jnp.float32),
              jax.ShapeDtypeStruct((8 * N_PAD,), jnp.float32)),
    mesh=plsc.VectorSubcoreMesh(core_axis_name="c", subcore_axis_name="s"),
    scratch_types=(
        [pltpu.VMEM((CHB,), jnp.int32)]
        + [pltpu.VMEM((CHB,), jnp.float32)] * 8
        + [pltpu.VMEM((NPT,), jnp.float32)]
        + [pltpu.VMEM_SHARED((N_PAD,), jnp.float32)] * 8
    ),
)(_pass_b)


def _sum_t_body(a_ref, b_ref, o_ref):
    o_ref[...] = a_ref[...] + b_ref[...]


def _tc_sum_t(p0, p1):
    tcw = 5888
    return pl.pallas_call(
        _sum_t_body,
        grid=(N_PAD // tcw,),
        in_specs=[pl.BlockSpec((8, tcw), lambda i: (0, i)),
                  pl.BlockSpec((8, tcw), lambda i: (0, i))],
        out_specs=pl.BlockSpec((8, tcw), lambda i: (0, i)),
        out_shape=jax.ShapeDtypeStruct((8, N_PAD), jnp.float32),
    )(p0, p1)


@jax.jit
def kernel(node_irreps, edge_index, sh_edge_features_0, sh_edge_features_1, W):
    edge_flat = edge_index.reshape(2 * N_EDGES)
    h = sh_edge_features_0.reshape(N_EDGES)
    r0 = sh_edge_features_1[:, 0]
    r1 = sh_edge_features_1[:, 1]
    r2 = sh_edge_features_1[:, 2]
    node_t = jnp.pad(node_irreps, ((0, N_PAD - N_NODES), (0, 0))).T

    s3 = 1.0 / math.sqrt(3.0)
    s6 = 1.0 / math.sqrt(6.0)
    A, B, C, D, F = W[0], W[1] * s3, W[2] * s3, W[3] * s3, W[4] * s6
    wflat = jnp.concatenate([
        jnp.stack([A[0, 0], A[1, 0], D[0, 0], D[1, 0],
                   A[0, 1], A[1, 1], D[0, 1], D[1, 1],
                   B[0, 0], B[1, 0], F[0, 0], F[1, 0],
                   B[0, 1], B[1, 1], F[0, 1], F[1, 1],
                   C[0, 0], C[1, 0], C[0, 1], C[1, 1]]),
        jnp.zeros((12,), jnp.float32),
    ])

    y = _pass_a_call(node_t.reshape(8 * N_PAD), edge_flat, h, r0, r1, r2,
                     wflat)
    p0, p1 = _pass_b_call(edge_flat, y)
    summed = _tc_sum_t(p0.reshape(8, N_PAD), p1.reshape(8, N_PAD))
    return summed[:, :N_NODES].T


# speedup vs baseline: 14.0713x; 1.4215x over previous
import functools
import math

import jax
import jax.numpy as jnp
from jax import lax
from jax.experimental import pallas as pl
from jax.experimental.pallas import tpu as pltpu
from jax.experimental.pallas import tpu_sc as plsc

N_NODES = 100000
N_EDGES = 3200000

NC = 2
NS = 16
NW = NC * NS
EPW = N_EDGES // NW
CH = 2000
NCHUNK = EPW // CH
CHB = 4000
NCHUNKB = EPW // CHB
N_PAD = 100096
NPT = N_PAD // NS


def _pass_a(*refs):
    (node_t_hbm, src_hbm, h_hbm, r0_hbm, r1_hbm, r2_hbm, w_hbm,
     y_hbm) = refs[:8]
    src_v = refs[8:10]
    h_v = refs[10:12]
    r_v = [refs[12:15], refs[15:18]]
    x_v = [refs[18:26], refs[26:34]]
    y_v = refs[34:42]
    w_v, stage_v = refs[42:44]
    nt = refs[44:52]
    lsem, gsem, wsem = refs[52:55]

    cid = lax.axis_index("c")
    sid = lax.axis_index("s")
    wid = cid * NS + sid

    rowbase = sid * NPT
    for c in range(8):
        pltpu.sync_copy(node_t_hbm.at[pl.ds(c * N_PAD + rowbase, NPT)],
                        stage_v)
        pltpu.sync_copy(stage_v, nt[c].at[pl.ds(rowbase, NPT)])

    pltpu.sync_copy(w_hbm, w_v)
    plsc.subcore_barrier()

    w_lo = w_v[pl.ds(0, 16)]
    w_hi = w_v[pl.ds(16, 16)]
    (a00, a10, d00, d10, a01, a11, d01, d11,
     b00, b10, f00, f10, b01, b11, f01, f11) = [w_lo[k] for k in range(16)]
    c00, c10, c01, c11 = [w_hi[k] for k in range(4)]

    ebase = wid * EPW

    def chunk_base(cc):
        return ebase + jnp.minimum(cc, NCHUNK - 1) * CH

    def lin_group(b, cc, mk):
        base = chunk_base(cc)
        return [mk(src_hbm.at[pl.ds(base, CH)], src_v[b], lsem),
                mk(h_hbm.at[pl.ds(base, CH)], h_v[b], lsem),
                mk(r0_hbm.at[pl.ds(base, CH)], r_v[b][0], lsem),
                mk(r1_hbm.at[pl.ds(base, CH)], r_v[b][1], lsem),
                mk(r2_hbm.at[pl.ds(base, CH)], r_v[b][2], lsem)]

    def issue_lin(b, cc):
        lin_group(b, cc, pltpu.async_copy)

    def wait_lin(b, cc):
        for d in lin_group(b, cc, pltpu.make_async_copy):
            d.wait()

    def issue_gathers(b):
        for c in range(8):
            pltpu.async_copy(nt[c].at[src_v[b]], x_v[b][c], gsem)

    def wait_gathers(b):
        dummy = h_hbm.at[pl.ds(ebase, CH)]
        for c in range(8):
            pltpu.make_async_copy(dummy, x_v[b][c], gsem).wait()

    def issue_ywr(cc):
        base = chunk_base(cc)
        for c in range(8):
            pltpu.async_copy(y_v[c], y_hbm.at[pl.ds(c * N_EDGES + base, CH)],
                             wsem)

    def wait_ywr():
        for c in range(8):
            pltpu.make_async_copy(y_v[c],
                                  y_hbm.at[pl.ds(c * N_EDGES + ebase, CH)],
                                  wsem).wait()

    def compute(b):
        def edge_body(i, _):
            sl = pl.ds(i * 16, 16)
            h = h_v[b][sl]
            r0, r1, r2 = r_v[b][0][sl], r_v[b][1][sl], r_v[b][2][sl]
            a0, a1 = x_v[b][0][sl], x_v[b][1][sl]
            u0, u1, u2 = x_v[b][2][sl], x_v[b][3][sl], x_v[b][4][sl]
            v0, v1, v2 = x_v[b][5][sl], x_v[b][6][sl], x_v[b][7][sl]

            t0 = h * a0
            t1 = h * a1
            dot_u = r0 * u0 + r1 * u1 + r2 * u2
            dot_v = r0 * v0 + r1 * v1 + r2 * v2
            y_v[0][sl] = a00 * t0 + a10 * t1 + d00 * dot_u + d10 * dot_v
            y_v[1][sl] = a01 * t0 + a11 * t1 + d01 * dot_u + d11 * dot_v

            hu0, hu1, hu2 = h * u0, h * u1, h * u2
            hv0, hv1, hv2 = h * v0, h * v1, h * v2
            ca = c00 * a0 + c10 * a1
            cb = c01 * a0 + c11 * a1
            cu0 = r1 * u2 - r2 * u1
            cu1 = r2 * u0 - r0 * u2
            cu2 = r0 * u1 - r1 * u0
            cv0 = r1 * v2 - r2 * v1
            cv1 = r2 * v0 - r0 * v2
            cv2 = r0 * v1 - r1 * v0

            y_v[2][sl] = (b00 * hu0 + b10 * hv0 + ca * r0
                             + f00 * cu0 + f10 * cv0)
            y_v[3][sl] = (b00 * hu1 + b10 * hv1 + ca * r1
                             + f00 * cu1 + f10 * cv1)
            y_v[4][sl] = (b00 * hu2 + b10 * hv2 + ca * r2
                             + f00 * cu2 + f10 * cv2)
            y_v[5][sl] = (b01 * hu0 + b11 * hv0 + cb * r0
                             + f01 * cu0 + f11 * cv0)
            y_v[6][sl] = (b01 * hu1 + b11 * hv1 + cb * r1
                             + f01 * cu1 + f11 * cv1)
            y_v[7][sl] = (b01 * hu2 + b11 * hv2 + cb * r2
                             + f01 * cu2 + f11 * cv2)
            return 0

        lax.fori_loop(0, CH // 16, edge_body, 0)

    issue_lin(0, 0)
    wait_lin(0, 0)
    issue_gathers(0)
    issue_lin(1, 1)

    def pair_body(k, _):
        c0 = 2 * k
        wait_lin(1, c0 + 1)
        issue_gathers(1)

        wait_gathers(0)

        @pl.when(k > 0)
        def _():
            wait_ywr()

        compute(0)
        issue_ywr(c0)
        issue_lin(0, c0 + 2)
        wait_lin(0, c0 + 2)
        issue_gathers(0)

        wait_gathers(1)
        wait_ywr()
        compute(1)
        issue_ywr(c0 + 1)
        issue_lin(1, c0 + 3)
        return 0

    lax.fori_loop(0, NCHUNK // 2, pair_body, 0)

    wait_lin(1, NCHUNK + 1)
    wait_gathers(0)
    wait_ywr()


_pass_a_call = functools.partial(
    pl.kernel,
    out_type=jax.ShapeDtypeStruct((8 * N_EDGES,), jnp.float32),
    mesh=plsc.VectorSubcoreMesh(core_axis_name="c", subcore_axis_name="s"),
    scratch_types=(
        [pltpu.VMEM((CH,), jnp.int32)] * 2
        + [pltpu.VMEM((CH,), jnp.float32)] * 2
        + [pltpu.VMEM((CH,), jnp.float32)] * 6
        + [pltpu.VMEM((CH,), jnp.float32)] * 16
        + [pltpu.VMEM((CH,), jnp.float32)] * 8
        + [pltpu.VMEM((32,), jnp.float32)]
        + [pltpu.VMEM((NPT,), jnp.float32)]
        + [pltpu.VMEM_SHARED((N_PAD,), jnp.float32)] * 8
        + [pltpu.SemaphoreType.DMA] * 3
    ),
)(_pass_a)


def _pass_b(*refs):
    (tgt_hbm, y_hbm, out0_hbm, out1_hbm) = refs[:4]
    tgt_v = refs[4]
    y_v = refs[5:13]
    stage_v = refs[13]
    acc = refs[14:22]

    cid = lax.axis_index("c")
    sid = lax.axis_index("s")
    wid = cid * NS + sid
    rowbase = sid * NPT

    def zfill_body(i, _):
        stage_v[pl.ds(i * 16, 16)] = jnp.zeros((16,), jnp.float32)
        return 0

    lax.fori_loop(0, NPT // 16, zfill_body, 0)
    for c in range(8):
        pltpu.sync_copy(stage_v, acc[c].at[pl.ds(rowbase, NPT)])
    plsc.subcore_barrier()

    def chunk_body(cc, _):
        base = wid * EPW + cc * CHB
        pltpu.sync_copy(tgt_hbm.at[pl.ds(N_EDGES + base, CHB)], tgt_v)
        for c in range(8):
            pltpu.sync_copy(y_hbm.at[pl.ds(c * N_EDGES + base, CHB)], y_v[c])
        for c in range(8):
            pltpu.sync_copy(y_v[c], acc[c].at[tgt_v], add=True)
        return 0

    lax.fori_loop(0, NCHUNKB, chunk_body, 0)
    plsc.subcore_barrier()

    for c in range(8):
        pltpu.sync_copy(acc[c].at[pl.ds(rowbase, NPT)], stage_v)

        @pl.when(cid == 0)
        def _(c=c):
            pltpu.sync_copy(stage_v,
                            out0_hbm.at[pl.ds(c * N_PAD + rowbase, NPT)])

        @pl.when(cid == 1)
        def _(c=c):
            pltpu.sync_copy(stage_v,
                            out1_hbm.at[pl.ds(c * N_PAD + rowbase, NPT)])


_pass_b_call = functools.partial(
    pl.kernel,
    out_type=(jax.ShapeDtypeStruct((8 * N_PAD,), jnp.float32),
              jax.ShapeDtypeStruct((8 * N_PAD,), jnp.float32)),
    mesh=plsc.VectorSubcoreMesh(core_axis_name="c", subcore_axis_name="s"),
    scratch_types=(
        [pltpu.VMEM((CHB,), jnp.int32)]
        + [pltpu.VMEM((CHB,), jnp.float32)] * 8
        + [pltpu.VMEM((NPT,), jnp.float32)]
        + [pltpu.VMEM_SHARED((N_PAD,), jnp.float32)] * 8
    ),
)(_pass_b)


def _sum_t_body(a_ref, b_ref, o_ref):
    o_ref[...] = a_ref[...] + b_ref[...]


def _tc_sum_t(p0, p1):
    tcw = 5888
    return pl.pallas_call(
        _sum_t_body,
        grid=(N_PAD // tcw,),
        in_specs=[pl.BlockSpec((8, tcw), lambda i: (0, i)),
                  pl.BlockSpec((8, tcw), lambda i: (0, i))],
        out_specs=pl.BlockSpec((8, tcw), lambda i: (0, i)),
        out_shape=jax.ShapeDtypeStruct((8, N_PAD), jnp.float32),
    )(p0, p1)


@jax.jit
def kernel(node_irreps, edge_index, sh_edge_features_0, sh_edge_features_1, W):
    edge_flat = edge_index.reshape(2 * N_EDGES)
    h = sh_edge_features_0.reshape(N_EDGES)
    r0 = sh_edge_features_1[:, 0]
    r1 = sh_edge_features_1[:, 1]
    r2 = sh_edge_features_1[:, 2]
    node_t = jnp.pad(node_irreps, ((0, N_PAD - N_NODES), (0, 0))).T

    s3 = 1.0 / math.sqrt(3.0)
    s6 = 1.0 / math.sqrt(6.0)
    A, B, C, D, F = W[0], W[1] * s3, W[2] * s3, W[3] * s3, W[4] * s6
    wflat = jnp.concatenate([
        jnp.stack([A[0, 0], A[1, 0], D[0, 0], D[1, 0],
                   A[0, 1], A[1, 1], D[0, 1], D[1, 1],
                   B[0, 0], B[1, 0], F[0, 0], F[1, 0],
                   B[0, 1], B[1, 1], F[0, 1], F[1, 1],
                   C[0, 0], C[1, 0], C[0, 1], C[1, 1]]),
        jnp.zeros((12,), jnp.float32),
    ])

    y = _pass_a_call(node_t.reshape(8 * N_PAD), edge_flat, h, r0, r1, r2,
                     wflat)
    p0, p1 = _pass_b_call(edge_flat, y)
    summed = _tc_sum_t(p0.reshape(8, N_PAD), p1.reshape(8, N_PAD))
    return summed[:, :N_NODES].T


# speedup vs baseline: 17.7590x; 1.2621x over previous
import functools
import math

import jax
import jax.numpy as jnp
from jax import lax
from jax.experimental import pallas as pl
from jax.experimental.pallas import tpu as pltpu
from jax.experimental.pallas import tpu_sc as plsc

N_NODES = 100000
N_EDGES = 3200000

NC = 2
NS = 16
NW = NC * NS
EPW = N_EDGES // NW
CH = 2000
NCHUNK = EPW // CH
CHB = 2000
NCHUNKB = EPW // CHB
N_PAD = 100096
NPT = N_PAD // NS


def _pass_a(*refs):
    (node_t_hbm, src_hbm, h_hbm, r0_hbm, r1_hbm, r2_hbm, w_hbm,
     y_hbm) = refs[:8]
    src_v = refs[8:10]
    h_v = refs[10:12]
    r_v = [refs[12:15], refs[15:18]]
    x_v = [refs[18:26], refs[26:34]]
    y_v = refs[34:42]
    w_v, stage_v = refs[42:44]
    nt = refs[44:52]
    lsem, gsem, wsem = refs[52:55]

    cid = lax.axis_index("c")
    sid = lax.axis_index("s")
    wid = cid * NS + sid

    rowbase = sid * NPT
    for c in range(8):
        pltpu.sync_copy(node_t_hbm.at[pl.ds(c * N_PAD + rowbase, NPT)],
                        stage_v)
        pltpu.sync_copy(stage_v, nt[c].at[pl.ds(rowbase, NPT)])

    pltpu.sync_copy(w_hbm, w_v)
    plsc.subcore_barrier()

    w_lo = w_v[pl.ds(0, 16)]
    w_hi = w_v[pl.ds(16, 16)]
    (a00, a10, d00, d10, a01, a11, d01, d11,
     b00, b10, f00, f10, b01, b11, f01, f11) = [w_lo[k] for k in range(16)]
    c00, c10, c01, c11 = [w_hi[k] for k in range(4)]

    ebase = wid * EPW

    def chunk_base(cc):
        return ebase + jnp.minimum(cc, NCHUNK - 1) * CH

    def lin_group(b, cc, mk):
        base = chunk_base(cc)
        return [mk(src_hbm.at[pl.ds(base, CH)], src_v[b], lsem),
                mk(h_hbm.at[pl.ds(base, CH)], h_v[b], lsem),
                mk(r0_hbm.at[pl.ds(base, CH)], r_v[b][0], lsem),
                mk(r1_hbm.at[pl.ds(base, CH)], r_v[b][1], lsem),
                mk(r2_hbm.at[pl.ds(base, CH)], r_v[b][2], lsem)]

    def issue_lin(b, cc):
        lin_group(b, cc, pltpu.async_copy)

    def wait_lin(b, cc):
        for d in lin_group(b, cc, pltpu.make_async_copy):
            d.wait()

    def issue_gathers(b):
        for c in range(8):
            pltpu.async_copy(nt[c].at[src_v[b]], x_v[b][c], gsem)

    def wait_gathers(b):
        dummy = h_hbm.at[pl.ds(ebase, CH)]
        for c in range(8):
            pltpu.make_async_copy(dummy, x_v[b][c], gsem).wait()

    def issue_ywr(cc):
        base = chunk_base(cc)
        for c in range(8):
            pltpu.async_copy(y_v[c], y_hbm.at[pl.ds(c * N_EDGES + base, CH)],
                             wsem)

    def wait_ywr():
        for c in range(8):
            pltpu.make_async_copy(y_v[c],
                                  y_hbm.at[pl.ds(c * N_EDGES + ebase, CH)],
                                  wsem).wait()

    def compute(b):
        def edge_body(i, _):
            sl = pl.ds(i * 16, 16)
            h = h_v[b][sl]
            r0, r1, r2 = r_v[b][0][sl], r_v[b][1][sl], r_v[b][2][sl]
            a0, a1 = x_v[b][0][sl], x_v[b][1][sl]
            u0, u1, u2 = x_v[b][2][sl], x_v[b][3][sl], x_v[b][4][sl]
            v0, v1, v2 = x_v[b][5][sl], x_v[b][6][sl], x_v[b][7][sl]

            t0 = h * a0
            t1 = h * a1
            dot_u = r0 * u0 + r1 * u1 + r2 * u2
            dot_v = r0 * v0 + r1 * v1 + r2 * v2
            y_v[0][sl] = a00 * t0 + a10 * t1 + d00 * dot_u + d10 * dot_v
            y_v[1][sl] = a01 * t0 + a11 * t1 + d01 * dot_u + d11 * dot_v

            hu0, hu1, hu2 = h * u0, h * u1, h * u2
            hv0, hv1, hv2 = h * v0, h * v1, h * v2
            ca = c00 * a0 + c10 * a1
            cb = c01 * a0 + c11 * a1
            cu0 = r1 * u2 - r2 * u1
            cu1 = r2 * u0 - r0 * u2
            cu2 = r0 * u1 - r1 * u0
            cv0 = r1 * v2 - r2 * v1
            cv1 = r2 * v0 - r0 * v2
            cv2 = r0 * v1 - r1 * v0

            y_v[2][sl] = (b00 * hu0 + b10 * hv0 + ca * r0
                             + f00 * cu0 + f10 * cv0)
            y_v[3][sl] = (b00 * hu1 + b10 * hv1 + ca * r1
                             + f00 * cu1 + f10 * cv1)
            y_v[4][sl] = (b00 * hu2 + b10 * hv2 + ca * r2
                             + f00 * cu2 + f10 * cv2)
            y_v[5][sl] = (b01 * hu0 + b11 * hv0 + cb * r0
                             + f01 * cu0 + f11 * cv0)
            y_v[6][sl] = (b01 * hu1 + b11 * hv1 + cb * r1
                             + f01 * cu1 + f11 * cv1)
            y_v[7][sl] = (b01 * hu2 + b11 * hv2 + cb * r2
                             + f01 * cu2 + f11 * cv2)
            return 0

        lax.fori_loop(0, CH // 16, edge_body, 0)

    issue_lin(0, 0)
    wait_lin(0, 0)
    issue_gathers(0)
    issue_lin(1, 1)

    def pair_body(k, _):
        c0 = 2 * k
        wait_lin(1, c0 + 1)
        issue_gathers(1)

        wait_gathers(0)

        @pl.when(k > 0)
        def _():
            wait_ywr()

        compute(0)
        issue_ywr(c0)
        issue_lin(0, c0 + 2)
        wait_lin(0, c0 + 2)
        issue_gathers(0)

        wait_gathers(1)
        wait_ywr()
        compute(1)
        issue_ywr(c0 + 1)
        issue_lin(1, c0 + 3)
        return 0

    lax.fori_loop(0, NCHUNK // 2, pair_body, 0)

    wait_lin(1, NCHUNK + 1)
    wait_gathers(0)
    wait_ywr()


_pass_a_call = functools.partial(
    pl.kernel,
    out_type=jax.ShapeDtypeStruct((8 * N_EDGES,), jnp.float32),
    mesh=plsc.VectorSubcoreMesh(core_axis_name="c", subcore_axis_name="s"),
    scratch_types=(
        [pltpu.VMEM((CH,), jnp.int32)] * 2
        + [pltpu.VMEM((CH,), jnp.float32)] * 2
        + [pltpu.VMEM((CH,), jnp.float32)] * 6
        + [pltpu.VMEM((CH,), jnp.float32)] * 16
        + [pltpu.VMEM((CH,), jnp.float32)] * 8
        + [pltpu.VMEM((32,), jnp.float32)]
        + [pltpu.VMEM((NPT,), jnp.float32)]
        + [pltpu.VMEM_SHARED((N_PAD,), jnp.float32)] * 8
        + [pltpu.SemaphoreType.DMA] * 3
    ),
)(_pass_a)


def _pass_b(*refs):
    (tgt_hbm, y_hbm, out0_hbm, out1_hbm) = refs[:4]
    tgt_v = refs[4:6]
    y_v = [refs[6:14], refs[14:22]]
    stage_v = refs[22]
    acc = refs[23:31]
    lsem, asem = refs[31:33]

    cid = lax.axis_index("c")
    sid = lax.axis_index("s")
    wid = cid * NS + sid
    rowbase = sid * NPT

    def zfill_body(i, _):
        stage_v[pl.ds(i * 16, 16)] = jnp.zeros((16,), jnp.float32)
        return 0

    lax.fori_loop(0, NPT // 16, zfill_body, 0)
    for c in range(8):
        pltpu.sync_copy(stage_v, acc[c].at[pl.ds(rowbase, NPT)])
    plsc.subcore_barrier()

    ebase = wid * EPW

    def chunk_base(cc):
        return ebase + jnp.minimum(cc, NCHUNKB - 1) * CHB

    def rd_group(b, cc, mk):
        base = chunk_base(cc)
        ds = [mk(tgt_hbm.at[pl.ds(N_EDGES + base, CHB)], tgt_v[b], lsem)]
        for c in range(8):
            ds.append(mk(y_hbm.at[pl.ds(c * N_EDGES + base, CHB)],
                         y_v[b][c], lsem))
        return ds

    def issue_rd(b, cc):
        rd_group(b, cc, pltpu.async_copy)

    def wait_rd(b, cc):
        for d in rd_group(b, cc, pltpu.make_async_copy):
            d.wait()

    def issue_adds(b):
        for c in range(8):
            pltpu.async_copy(y_v[b][c], acc[c].at[tgt_v[b]], asem, add=True)

    def wait_adds(b):
        dummy = y_hbm.at[pl.ds(ebase, CHB)]
        for c in range(8):
            pltpu.make_async_copy(dummy, y_v[b][c], asem).wait()

    issue_rd(0, 0)
    issue_rd(1, 1)

    def pair_body(k, _):
        c0 = 2 * k
        wait_rd(0, c0)
        issue_adds(0)
        wait_rd(1, c0 + 1)
        wait_adds(0)
        issue_rd(0, c0 + 2)
        issue_adds(1)
        wait_adds(1)
        issue_rd(1, c0 + 3)
        return 0

    lax.fori_loop(0, NCHUNKB // 2, pair_body, 0)

    wait_rd(0, NCHUNKB)
    wait_rd(1, NCHUNKB + 1)
    plsc.subcore_barrier()

    for c in range(8):
        pltpu.sync_copy(acc[c].at[pl.ds(rowbase, NPT)], stage_v)

        @pl.when(cid == 0)
        def _(c=c):
            pltpu.sync_copy(stage_v,
                            out0_hbm.at[pl.ds(c * N_PAD + rowbase, NPT)])

        @pl.when(cid == 1)
        def _(c=c):
            pltpu.sync_copy(stage_v,
                            out1_hbm.at[pl.ds(c * N_PAD + rowbase, NPT)])


_pass_b_call = functools.partial(
    pl.kernel,
    out_type=(jax.ShapeDtypeStruct((8 * N_PAD,), jnp.float32),
              jax.ShapeDtypeStruct((8 * N_PAD,), jnp.float32)),
    mesh=plsc.VectorSubcoreMesh(core_axis_name="c", subcore_axis_name="s"),
    scratch_types=(
        [pltpu.VMEM((CHB,), jnp.int32)] * 2
        + [pltpu.VMEM((CHB,), jnp.float32)] * 16
        + [pltpu.VMEM((NPT,), jnp.float32)]
        + [pltpu.VMEM_SHARED((N_PAD,), jnp.float32)] * 8
        + [pltpu.SemaphoreType.DMA] * 2
    ),
)(_pass_b)


def _sum_t_body(a_ref, b_ref, o_ref):
    o_ref[...] = a_ref[...] + b_ref[...]


def _tc_sum_t(p0, p1):
    tcw = 5888
    return pl.pallas_call(
        _sum_t_body,
        grid=(N_PAD // tcw,),
        in_specs=[pl.BlockSpec((8, tcw), lambda i: (0, i)),
                  pl.BlockSpec((8, tcw), lambda i: (0, i))],
        out_specs=pl.BlockSpec((8, tcw), lambda i: (0, i)),
        out_shape=jax.ShapeDtypeStruct((8, N_PAD), jnp.float32),
    )(p0, p1)


@jax.jit
def kernel(node_irreps, edge_index, sh_edge_features_0, sh_edge_features_1, W):
    edge_flat = edge_index.reshape(2 * N_EDGES)
    h = sh_edge_features_0.reshape(N_EDGES)
    r0 = sh_edge_features_1[:, 0]
    r1 = sh_edge_features_1[:, 1]
    r2 = sh_edge_features_1[:, 2]
    node_t = jnp.pad(node_irreps, ((0, N_PAD - N_NODES), (0, 0))).T

    s3 = 1.0 / math.sqrt(3.0)
    s6 = 1.0 / math.sqrt(6.0)
    A, B, C, D, F = W[0], W[1] * s3, W[2] * s3, W[3] * s3, W[4] * s6
    wflat = jnp.concatenate([
        jnp.stack([A[0, 0], A[1, 0], D[0, 0], D[1, 0],
                   A[0, 1], A[1, 1], D[0, 1], D[1, 1],
                   B[0, 0], B[1, 0], F[0, 0], F[1, 0],
                   B[0, 1], B[1, 1], F[0, 1], F[1, 1],
                   C[0, 0], C[1, 0], C[0, 1], C[1, 1]]),
        jnp.zeros((12,), jnp.float32),
    ])

    y = _pass_a_call(node_t.reshape(8 * N_PAD), edge_flat, h, r0, r1, r2,
                     wflat)
    p0, p1 = _pass_b_call(edge_flat, y)
    summed = _tc_sum_t(p0.reshape(8, N_PAD), p1.reshape(8, N_PAD))
    return summed[:, :N_NODES].T


# speedup vs baseline: 18.0852x; 1.0184x over previous
import functools
import math

import jax
import jax.numpy as jnp
from jax import lax
from jax.experimental import pallas as pl
from jax.experimental.pallas import tpu as pltpu
from jax.experimental.pallas import tpu_sc as plsc

N_NODES = 100000
N_EDGES = 3200000

NC = 2
NS = 16
NW = NC * NS
EPW = N_EDGES // NW
CH = 2000
NCHUNK = EPW // CH
CHB = 2000
NCHUNKB = EPW // CHB
N_PAD = 100096
NPT = N_PAD // NS
NPB = 6400
NPB_LAST = N_NODES - 15 * NPB


def _pass_a(*refs):
    (node_t_hbm, src_hbm, h_hbm, r0_hbm, r1_hbm, r2_hbm, w_hbm,
     y_hbm) = refs[:8]
    src_v = refs[8:10]
    h_v = refs[10:12]
    r_v = [refs[12:15], refs[15:18]]
    x_v = [refs[18:26], refs[26:34]]
    y_v = refs[34:42]
    w_v, stage_v = refs[42:44]
    nt = refs[44:52]
    lsem, gsem, wsem = refs[52:55]

    cid = lax.axis_index("c")
    sid = lax.axis_index("s")
    wid = cid * NS + sid

    rowbase = sid * NPT
    for c in range(8):
        pltpu.sync_copy(node_t_hbm.at[pl.ds(c * N_PAD + rowbase, NPT)],
                        stage_v)
        pltpu.sync_copy(stage_v, nt[c].at[pl.ds(rowbase, NPT)])

    pltpu.sync_copy(w_hbm, w_v)
    plsc.subcore_barrier()

    w_lo = w_v[pl.ds(0, 16)]
    w_hi = w_v[pl.ds(16, 16)]
    (a00, a10, d00, d10, a01, a11, d01, d11,
     b00, b10, f00, f10, b01, b11, f01, f11) = [w_lo[k] for k in range(16)]
    c00, c10, c01, c11 = [w_hi[k] for k in range(4)]

    ebase = wid * EPW

    def chunk_base(cc):
        return ebase + jnp.minimum(cc, NCHUNK - 1) * CH

    def lin_group(b, cc, mk):
        base = chunk_base(cc)
        return [mk(src_hbm.at[pl.ds(base, CH)], src_v[b], lsem),
                mk(h_hbm.at[pl.ds(base, CH)], h_v[b], lsem),
                mk(r0_hbm.at[pl.ds(base, CH)], r_v[b][0], lsem),
                mk(r1_hbm.at[pl.ds(base, CH)], r_v[b][1], lsem),
                mk(r2_hbm.at[pl.ds(base, CH)], r_v[b][2], lsem)]

    def issue_lin(b, cc):
        lin_group(b, cc, pltpu.async_copy)

    def wait_lin(b, cc):
        for d in lin_group(b, cc, pltpu.make_async_copy):
            d.wait()

    def issue_gathers(b):
        for c in range(8):
            pltpu.async_copy(nt[c].at[src_v[b]], x_v[b][c], gsem)

    def wait_gathers(b):
        dummy = h_hbm.at[pl.ds(ebase, CH)]
        for c in range(8):
            pltpu.make_async_copy(dummy, x_v[b][c], gsem).wait()

    def issue_ywr(cc):
        base = chunk_base(cc)
        for c in range(8):
            pltpu.async_copy(y_v[c], y_hbm.at[pl.ds(c * N_EDGES + base, CH)],
                             wsem)

    def wait_ywr():
        for c in range(8):
            pltpu.make_async_copy(y_v[c],
                                  y_hbm.at[pl.ds(c * N_EDGES + ebase, CH)],
                                  wsem).wait()

    def compute(b):
        def edge_body(i, _):
            sl = pl.ds(i * 16, 16)
            h = h_v[b][sl]
            r0, r1, r2 = r_v[b][0][sl], r_v[b][1][sl], r_v[b][2][sl]
            a0, a1 = x_v[b][0][sl], x_v[b][1][sl]
            u0, u1, u2 = x_v[b][2][sl], x_v[b][3][sl], x_v[b][4][sl]
            v0, v1, v2 = x_v[b][5][sl], x_v[b][6][sl], x_v[b][7][sl]

            t0 = h * a0
            t1 = h * a1
            dot_u = r0 * u0 + r1 * u1 + r2 * u2
            dot_v = r0 * v0 + r1 * v1 + r2 * v2
            y_v[0][sl] = a00 * t0 + a10 * t1 + d00 * dot_u + d10 * dot_v
            y_v[1][sl] = a01 * t0 + a11 * t1 + d01 * dot_u + d11 * dot_v

            hu0, hu1, hu2 = h * u0, h * u1, h * u2
            hv0, hv1, hv2 = h * v0, h * v1, h * v2
            ca = c00 * a0 + c10 * a1
            cb = c01 * a0 + c11 * a1
            cu0 = r1 * u2 - r2 * u1
            cu1 = r2 * u0 - r0 * u2
            cu2 = r0 * u1 - r1 * u0
            cv0 = r1 * v2 - r2 * v1
            cv1 = r2 * v0 - r0 * v2
            cv2 = r0 * v1 - r1 * v0

            y_v[2][sl] = (b00 * hu0 + b10 * hv0 + ca * r0
                             + f00 * cu0 + f10 * cv0)
            y_v[3][sl] = (b00 * hu1 + b10 * hv1 + ca * r1
                             + f00 * cu1 + f10 * cv1)
            y_v[4][sl] = (b00 * hu2 + b10 * hv2 + ca * r2
                             + f00 * cu2 + f10 * cv2)
            y_v[5][sl] = (b01 * hu0 + b11 * hv0 + cb * r0
                             + f01 * cu0 + f11 * cv0)
            y_v[6][sl] = (b01 * hu1 + b11 * hv1 + cb * r1
                             + f01 * cu1 + f11 * cv1)
            y_v[7][sl] = (b01 * hu2 + b11 * hv2 + cb * r2
                             + f01 * cu2 + f11 * cv2)
            return 0

        lax.fori_loop(0, CH // 16, edge_body, 0)

    issue_lin(0, 0)
    wait_lin(0, 0)
    issue_gathers(0)
    issue_lin(1, 1)

    def pair_body(k, _):
        c0 = 2 * k
        wait_lin(1, c0 + 1)
        issue_gathers(1)

        wait_gathers(0)

        @pl.when(k > 0)
        def _():
            wait_ywr()

        compute(0)
        issue_ywr(c0)
        issue_lin(0, c0 + 2)
        wait_lin(0, c0 + 2)
        issue_gathers(0)

        wait_gathers(1)
        wait_ywr()
        compute(1)
        issue_ywr(c0 + 1)
        issue_lin(1, c0 + 3)
        return 0

    lax.fori_loop(0, NCHUNK // 2, pair_body, 0)

    wait_lin(1, NCHUNK + 1)
    wait_gathers(0)
    wait_ywr()


_pass_a_call = functools.partial(
    pl.kernel,
    out_type=jax.ShapeDtypeStruct((8 * N_EDGES,), jnp.float32),
    mesh=plsc.VectorSubcoreMesh(core_axis_name="c", subcore_axis_name="s"),
    scratch_types=(
        [pltpu.VMEM((CH,), jnp.int32)] * 2
        + [pltpu.VMEM((CH,), jnp.float32)] * 2
        + [pltpu.VMEM((CH,), jnp.float32)] * 6
        + [pltpu.VMEM((CH,), jnp.float32)] * 16
        + [pltpu.VMEM((CH,), jnp.float32)] * 8
        + [pltpu.VMEM((32,), jnp.float32)]
        + [pltpu.VMEM((NPT,), jnp.float32)]
        + [pltpu.VMEM_SHARED((N_PAD,), jnp.float32)] * 8
        + [pltpu.SemaphoreType.DMA] * 3
    ),
)(_pass_a)


def _pass_b(*refs):
    (tgt_hbm, y_hbm, out0_hbm, out1_hbm) = refs[:4]
    tgt_v = refs[4:6]
    y_v = [refs[6:14], refs[14:22]]
    stage_v = refs[22]
    acc = refs[23:31]
    lsem, asem = refs[31:33]

    cid = lax.axis_index("c")
    sid = lax.axis_index("s")
    wid = cid * NS + sid
    rowbase = sid * NPB
    nrows = jnp.where(sid == NS - 1, NPB_LAST, NPB)

    def zfill_body(i, _):
        stage_v[pl.ds(i * 16, 16)] = jnp.zeros((16,), jnp.float32)
        return 0

    lax.fori_loop(0, NPB // 16, zfill_body, 0)
    for c in range(8):
        @pl.when(sid < NS - 1)
        def _(c=c):
            pltpu.sync_copy(stage_v, acc[c].at[pl.ds(rowbase, NPB)])

        @pl.when(sid == NS - 1)
        def _(c=c):
            pltpu.sync_copy(stage_v.at[pl.ds(0, NPB_LAST)],
                            acc[c].at[pl.ds(rowbase, NPB_LAST)])
    plsc.subcore_barrier()

    ebase = wid * EPW

    def chunk_base(cc):
        return ebase + jnp.minimum(cc, NCHUNKB - 1) * CHB

    def rd_group(b, cc, mk):
        base = chunk_base(cc)
        ds = [mk(tgt_hbm.at[pl.ds(N_EDGES + base, CHB)], tgt_v[b], lsem)]
        for c in range(8):
            ds.append(mk(y_hbm.at[pl.ds(c * N_EDGES + base, CHB)],
                         y_v[b][c], lsem))
        return ds

    def issue_rd(b, cc):
        rd_group(b, cc, pltpu.async_copy)

    def wait_rd(b, cc):
        for d in rd_group(b, cc, pltpu.make_async_copy):
            d.wait()

    def issue_adds(b):
        for c in range(8):
            pltpu.async_copy(y_v[b][c], acc[c].at[tgt_v[b]], asem, add=True)

    def wait_adds(b):
        dummy = y_hbm.at[pl.ds(ebase, CHB)]
        for c in range(8):
            pltpu.make_async_copy(dummy, y_v[b][c], asem).wait()

    issue_rd(0, 0)
    issue_rd(1, 1)

    def pair_body(k, _):
        c0 = 2 * k
        wait_rd(0, c0)
        issue_adds(0)
        wait_rd(1, c0 + 1)
        wait_adds(0)
        issue_rd(0, c0 + 2)
        issue_adds(1)
        wait_adds(1)
        issue_rd(1, c0 + 3)
        return 0

    lax.fori_loop(0, NCHUNKB // 2, pair_body, 0)

    wait_rd(0, NCHUNKB)
    wait_rd(1, NCHUNKB + 1)
    plsc.subcore_barrier()

    for c in range(8):
        @pl.when(sid < NS - 1)
        def _(c=c):
            pltpu.sync_copy(acc[c].at[pl.ds(rowbase, NPB)], stage_v)

            @pl.when(cid == 0)
            def _():
                pltpu.sync_copy(stage_v,
                                out0_hbm.at[pl.ds(c * N_NODES + rowbase, NPB)])

            @pl.when(cid == 1)
            def _():
                pltpu.sync_copy(stage_v,
                                out1_hbm.at[pl.ds(c * N_NODES + rowbase, NPB)])

        @pl.when(sid == NS - 1)
        def _(c=c):
            pltpu.sync_copy(acc[c].at[pl.ds(rowbase, NPB_LAST)],
                            stage_v.at[pl.ds(0, NPB_LAST)])

            @pl.when(cid == 0)
            def _():
                pltpu.sync_copy(stage_v.at[pl.ds(0, NPB_LAST)],
                                out0_hbm.at[pl.ds(c * N_NODES + rowbase,
                                                  NPB_LAST)])

            @pl.when(cid == 1)
            def _():
                pltpu.sync_copy(stage_v.at[pl.ds(0, NPB_LAST)],
                                out1_hbm.at[pl.ds(c * N_NODES + rowbase,
                                                  NPB_LAST)])


_pass_b_call = functools.partial(
    pl.kernel,
    out_type=(jax.ShapeDtypeStruct((8 * N_NODES,), jnp.float32),
              jax.ShapeDtypeStruct((8 * N_NODES,), jnp.float32)),
    mesh=plsc.VectorSubcoreMesh(core_axis_name="c", subcore_axis_name="s"),
    scratch_types=(
        [pltpu.VMEM((CHB,), jnp.int32)] * 2
        + [pltpu.VMEM((CHB,), jnp.float32)] * 16
        + [pltpu.VMEM((NPB,), jnp.float32)]
        + [pltpu.VMEM_SHARED((N_NODES,), jnp.float32)] * 8
        + [pltpu.SemaphoreType.DMA] * 2
    ),
)(_pass_b)


def _sum_t_body(a_ref, b_ref, o_ref):
    o_ref[...] = a_ref[...] + b_ref[...]


def _tc_sum_t(p0, p1):
    return pl.pallas_call(
        _sum_t_body,
        out_shape=jax.ShapeDtypeStruct((8 * N_NODES,), jnp.float32),
    )(p0, p1)


@jax.jit
def kernel(node_irreps, edge_index, sh_edge_features_0, sh_edge_features_1, W):
    edge_flat = edge_index.reshape(2 * N_EDGES)
    h = sh_edge_features_0.reshape(N_EDGES)
    r0 = sh_edge_features_1[:, 0]
    r1 = sh_edge_features_1[:, 1]
    r2 = sh_edge_features_1[:, 2]
    node_t = jnp.pad(node_irreps, ((0, N_PAD - N_NODES), (0, 0))).T

    s3 = 1.0 / math.sqrt(3.0)
    s6 = 1.0 / math.sqrt(6.0)
    A, B, C, D, F = W[0], W[1] * s3, W[2] * s3, W[3] * s3, W[4] * s6
    wflat = jnp.concatenate([
        jnp.stack([A[0, 0], A[1, 0], D[0, 0], D[1, 0],
                   A[0, 1], A[1, 1], D[0, 1], D[1, 1],
                   B[0, 0], B[1, 0], F[0, 0], F[1, 0],
                   B[0, 1], B[1, 1], F[0, 1], F[1, 1],
                   C[0, 0], C[1, 0], C[0, 1], C[1, 1]]),
        jnp.zeros((12,), jnp.float32),
    ])

    y = _pass_a_call(node_t.reshape(8 * N_PAD), edge_flat, h, r0, r1, r2,
                     wflat)
    p0, p1 = _pass_b_call(edge_flat, y)
    return _tc_sum_t(p0, p1).reshape(8, N_NODES).T
